# baseline (speedup 1.0000x reference)
import jax
import jax.numpy as jnp
from jax import lax
from jax.experimental import pallas as pl
from jax.experimental.pallas import tpu as pltpu
from jax.experimental.pallas import tpu_sc as plsc

NUM_ENTITY = 1000000
NUM_RELATION = 1000
EMBED_DIM = 64
MARGIN = 1.0
NEG_SAMPLES = 2
BATCH = 4096
PACK = 2 * EMBED_DIM

_INFO = plsc.get_sparse_core_info()
_NC = _INFO.num_cores
_NS = _INFO.num_subcores
_L = _INFO.num_lanes
_NW = _NC * _NS
_PPW = BATCH // _NW
_NPW = _PPW * NEG_SAMPLES
_GROUPS = _PPW // _L


def _tec_body(pos_h, pos_r, pos_t, neg_h, neg_r, neg_t, ent2, rel2, out,
              i_ph, i_pt, i_pr, i_nh0, i_nh1, i_nt0, i_nt1, i_nr0, i_nr1,
              g_ph, g_pt, g_pr, g_nh0, g_nh1, g_nt0, g_nt1, g_nr0, g_nr1,
              bA0, bA1, bA2, bB0, bB1, bB2,
              shared, tmp, stage, semA, semB):
    cid = lax.axis_index("c")
    sid = lax.axis_index("s")
    wid = sid * _NC + cid
    pbase = wid * _PPW
    nbase = wid * _NPW

    cps = [
        pltpu.async_copy(pos_h.at[pl.ds(pbase, _PPW)], i_ph, semA),
        pltpu.async_copy(pos_t.at[pl.ds(pbase, _PPW)], i_pt, semA),
        pltpu.async_copy(pos_r.at[pl.ds(pbase, _PPW)], i_pr, semA),
        pltpu.async_copy(neg_h.at[pl.ds(nbase, _PPW)], i_nh0, semA),
        pltpu.async_copy(neg_h.at[pl.ds(nbase + _PPW, _PPW)], i_nh1, semA),
        pltpu.async_copy(neg_t.at[pl.ds(nbase, _PPW)], i_nt0, semA),
        pltpu.async_copy(neg_t.at[pl.ds(nbase + _PPW, _PPW)], i_nt1, semA),
        pltpu.async_copy(neg_r.at[pl.ds(nbase, _PPW)], i_nr0, semA),
        pltpu.async_copy(neg_r.at[pl.ds(nbase + _PPW, _PPW)], i_nr1, semA),
    ]
    for c in cps:
        c.wait()

    for i_ref, g_ref in ((i_ph, g_ph), (i_pt, g_pt), (i_pr, g_pr),
                         (i_nh0, g_nh0), (i_nh1, g_nh1), (i_nt0, g_nt0),
                         (i_nt1, g_nt1), (i_nr0, g_nr0), (i_nr1, g_nr1)):
        def shift_body(c, _, i_ref=i_ref, g_ref=g_ref):
            sl = pl.ds(c * _L, _L)
            g_ref[sl] = lax.shift_right_logical(i_ref[sl], 1)
            return 0
        lax.fori_loop(0, _PPW // _L, shift_body, 0)

    a_cps = [
        pltpu.async_copy(ent2.at[g_ph], bA0, semA),
        pltpu.async_copy(ent2.at[g_pt], bA1, semA),
        pltpu.async_copy(rel2.at[g_pr], bA2, semA),
    ]
    b_cps = [
        pltpu.async_copy(ent2.at[g_nh0], bB0, semB),
        pltpu.async_copy(ent2.at[g_nt0], bB1, semB),
        pltpu.async_copy(rel2.at[g_nr0], bB2, semB),
    ]
    for c in a_cps:
        c.wait()

    lane = lax.iota(jnp.int32, _L)
    zero = jnp.zeros((_L,), jnp.float32)

    accs_p = []
    for g in range(_GROUPS):
        rows = g * _L + lane
        par_h = (i_ph[pl.ds(g * _L, _L)] & 1) * EMBED_DIM
        par_t = (i_pt[pl.ds(g * _L, _L)] & 1) * EMBED_DIM
        par_r = (i_pr[pl.ds(g * _L, _L)] & 1) * EMBED_DIM

        def dbody(d, ap, rows=rows, par_h=par_h, par_t=par_t, par_r=par_r):
            h = plsc.load_gather(bA0, [rows, par_h + d])
            t = plsc.load_gather(bA1, [rows, par_t + d])
            r = plsc.load_gather(bA2, [rows, par_r + d])
            return ap + jnp.abs(h + r - t)

        accs_p.append(lax.fori_loop(0, EMBED_DIM, dbody, zero))

    a2_cps = [
        pltpu.async_copy(ent2.at[g_nh1], bA0, semA),
        pltpu.async_copy(ent2.at[g_nt1], bA1, semA),
        pltpu.async_copy(rel2.at[g_nr1], bA2, semA),
    ]
    for c in b_cps:
        c.wait()

    loss_acc = zero

    def neg_groups(g_lo, g_hi, ih, it, ir, b0, b1, b2, loss_acc):
        for g in range(g_lo, g_hi):
            j0 = 2 * ((g - g_lo) * _L + lane)
            j1 = j0 + 1
            ph0 = (plsc.load_gather(ih, [j0]) & 1) * EMBED_DIM
            pt0 = (plsc.load_gather(it, [j0]) & 1) * EMBED_DIM
            pr0 = (plsc.load_gather(ir, [j0]) & 1) * EMBED_DIM
            ph1 = (plsc.load_gather(ih, [j1]) & 1) * EMBED_DIM
            pt1 = (plsc.load_gather(it, [j1]) & 1) * EMBED_DIM
            pr1 = (plsc.load_gather(ir, [j1]) & 1) * EMBED_DIM

            def dbody(d, carry, j0=j0, j1=j1, ph0=ph0, pt0=pt0, pr0=pr0,
                      ph1=ph1, pt1=pt1, pr1=pr1, b0=b0, b1=b1, b2=b2):
                a0, a1 = carry
                h0 = plsc.load_gather(b0, [j0, ph0 + d])
                t0 = plsc.load_gather(b1, [j0, pt0 + d])
                r0 = plsc.load_gather(b2, [j0, pr0 + d])
                a0 = a0 + jnp.abs(h0 + r0 - t0)
                h1 = plsc.load_gather(b0, [j1, ph1 + d])
                t1 = plsc.load_gather(b1, [j1, pt1 + d])
                r1 = plsc.load_gather(b2, [j1, pr1 + d])
                a1 = a1 + jnp.abs(h1 + r1 - t1)
                return a0, a1

            an0, an1 = lax.fori_loop(0, EMBED_DIM, dbody, (zero, zero))
            n_sc = (an0 + an1) * 0.5
            loss_acc = loss_acc + jnp.maximum(accs_p[g] - n_sc + MARGIN, 0.0)
        return loss_acc

    loss_acc = neg_groups(0, _GROUPS // 2,
                          i_nh0, i_nt0, i_nr0, bB0, bB1, bB2, loss_acc)
    for c in a2_cps:
        c.wait()
    loss_acc = neg_groups(_GROUPS // 2, _GROUPS,
                          i_nh1, i_nt1, i_nr1, bA0, bA1, bA2, loss_acc)

    stage[pl.ds(0, _L)] = loss_acc
    pltpu.sync_copy(stage.at[pl.ds(0, _L)], shared.at[sid, pl.ds(0, _L)])
    plsc.subcore_barrier()

    @pl.when(sid == 0)
    def _():
        pltpu.sync_copy(shared, tmp)
        acc = jnp.zeros((_L,), jnp.float32)
        for w in range(_NS):
            acc = acc + tmp[w, pl.ds(0, _L)]
        total = jnp.sum(acc)
        stage[pl.ds(0, _L)] = jnp.full((_L,), total, jnp.float32)
        pltpu.sync_copy(stage.at[pl.ds(0, _L)], out.at[cid, pl.ds(0, _L)])


_FULLB = NUM_ENTITY // 128
_PARTIAL_COLS = NUM_ENTITY - _FULLB * 128
_MAIN_ITERS = (_FULLB // _NW) // 2
_TAIL_W = _FULLB - _MAIN_ITERS * 2 * _NW


def _transpose_body(entT, tail2, t2, ib0, ib1, ob, rsm0, rsm1):
    cid = lax.axis_index("c")
    sid = lax.axis_index("s")
    wid = sid * _NC + cid
    lane = lax.iota(jnp.int32, _L)

    def rd(buf, i, sm):
        c0 = pl.multiple_of((i * _NW + wid) * 128, 128)
        return pltpu.async_copy(entT.at[:, pl.ds(c0, 128)],
                                buf.at[:, pl.ds(0, 128)], sm)

    def wait_rd(buf, sm):
        pltpu.make_async_copy(entT.at[:, pl.ds(0, 128)],
                              buf.at[:, pl.ds(0, 128)], sm).wait()

    def transpose_block(src, rows_out):
        @plsc.parallel_loop(0, EMBED_DIM, unroll=2)
        def _(p):
            col0 = jnp.full((_L,), 2 * p, jnp.int32)
            col1 = col0 + 1
            for c in range(EMBED_DIM // _L):
                rows = c * _L + lane
                ob[p, pl.ds(c * _L, _L)] = plsc.load_gather(src,
                                                            [rows, col0])
                ob[p, pl.ds(EMBED_DIM + c * _L, _L)] = plsc.load_gather(
                    src, [rows, col1])

    def wr(i, nrows):
        p0 = pl.multiple_of((i * _NW + wid) * 64, 64)
        pltpu.sync_copy(ob.at[pl.ds(0, nrows)], t2.at[pl.ds(p0, nrows)])

    rd(ib0, 0, rsm0)

    def pair(j, _):
        rd(ib1, 2 * j + 1, rsm1)
        wait_rd(ib0, rsm0)
        transpose_block(ib0, 64)
        wr(2 * j, 64)

        @pl.when(j < _MAIN_ITERS - 1)
        def _():
            rd(ib0, 2 * j + 2, rsm0)

        wait_rd(ib1, rsm1)
        transpose_block(ib1, 64)
        wr(2 * j + 1, 64)
        return 0

    lax.fori_loop(0, _MAIN_ITERS, pair, 0)

    @pl.when(wid < _TAIL_W)
    def _():
        bid = _MAIN_ITERS * 2 * _NW + wid
        c0 = pl.multiple_of(bid * 128, 128)
        pltpu.sync_copy(entT.at[:, pl.ds(c0, 128)], ib0.at[:, pl.ds(0, 128)])
        transpose_block(ib0, 64)
        p0 = pl.multiple_of(bid * 64, 64)
        pltpu.sync_copy(ob, t2.at[pl.ds(p0, 64)])

    @pl.when(wid == _TAIL_W)
    def _():
        nr = _PARTIAL_COLS // 2
        pltpu.sync_copy(tail2, ib0.at[pl.ds(0, nr), pl.ds(0, 128)])
        pltpu.sync_copy(ib0.at[pl.ds(0, nr), pl.ds(0, 128)],
                        t2.at[pl.ds(_FULLB * 64, nr)])


@jax.jit
def _transe_loss(pos_h, pos_r, pos_t, neg_h, neg_r, neg_t, entT, tail2, rel2):
    mesh = plsc.VectorSubcoreMesh(core_axis_name="c", subcore_axis_name="s")
    ent2 = pl.kernel(
        _transpose_body,
        out_type=jax.ShapeDtypeStruct((NUM_ENTITY // 2, PACK), jnp.float32),
        mesh=mesh,
        compiler_params=pltpu.CompilerParams(needs_layout_passes=False),
        scratch_types=[
            pltpu.VMEM((EMBED_DIM, 129), jnp.float32),
            pltpu.VMEM((EMBED_DIM, 129), jnp.float32),
            pltpu.VMEM((EMBED_DIM, PACK), jnp.float32),
            pltpu.SemaphoreType.DMA,
            pltpu.SemaphoreType.DMA,
        ],
    )(entT, tail2)
    idx_t = pltpu.VMEM((_PPW,), jnp.int32)
    buf_t = pltpu.VMEM((_PPW, PACK), jnp.float32)
    out = pl.kernel(
        _tec_body,
        out_type=jax.ShapeDtypeStruct((8, 128), jnp.float32),
        mesh=mesh,
        compiler_params=pltpu.CompilerParams(needs_layout_passes=False),
        scratch_types=[
            idx_t, idx_t, idx_t, idx_t, idx_t, idx_t, idx_t, idx_t, idx_t,
            idx_t, idx_t, idx_t, idx_t, idx_t, idx_t, idx_t, idx_t, idx_t,
            buf_t, buf_t, buf_t, buf_t, buf_t, buf_t,
            pltpu.VMEM_SHARED((_NS, 128), jnp.float32),
            pltpu.VMEM((_NS, 128), jnp.float32),
            pltpu.VMEM((128,), jnp.float32),
            pltpu.SemaphoreType.DMA,
            pltpu.SemaphoreType.DMA,
        ],
    )(pos_h, pos_r, pos_t, neg_h, neg_r, neg_t, ent2, rel2)
    return out[0, 0] + out[1, 0]


def kernel(pos_h, pos_r, pos_t, neg_h, neg_r, neg_t, entity_emb, relation_emb):
    rel2 = relation_emb.reshape(NUM_RELATION // 2, PACK)
    tail2 = entity_emb[_FULLB * 128:, :].reshape(_PARTIAL_COLS // 2, PACK)
    return _transe_loss(
        pos_h.astype(jnp.int32), pos_r.astype(jnp.int32),
        pos_t.astype(jnp.int32), neg_h.astype(jnp.int32),
        neg_r.astype(jnp.int32), neg_t.astype(jnp.int32),
        entity_emb.T, tail2, rel2)


# speedup vs baseline: 1.0976x; 1.0976x over previous
import jax
import jax.numpy as jnp
from jax import lax
from jax.experimental import pallas as pl
from jax.experimental.pallas import tpu as pltpu
from jax.experimental.pallas import tpu_sc as plsc

NUM_ENTITY = 1000000
NUM_RELATION = 1000
EMBED_DIM = 64
MARGIN = 1.0
NEG_SAMPLES = 2
BATCH = 4096
PACK = 2 * EMBED_DIM

_INFO = plsc.get_sparse_core_info()
_NC = _INFO.num_cores
_NS = _INFO.num_subcores
_L = _INFO.num_lanes
_NW = _NC * _NS
_PPW = BATCH // _NW
_NPW = _PPW * NEG_SAMPLES
_GROUPS = _PPW // _L


def _tec_body(pos_h, pos_r, pos_t, neg_h, neg_r, neg_t, ent2, rel2, out,
              i_ph, i_pt, i_pr, i_nh0, i_nh1, i_nt0, i_nt1, i_nr0, i_nr1,
              g_ph, g_pt, g_pr, g_nh0, g_nh1, g_nt0, g_nt1, g_nr0, g_nr1,
              bA0, bA1, bA2, bB0, bB1, bB2,
              shared, tmp, stage, semA, semB):
    cid = lax.axis_index("c")
    sid = lax.axis_index("s")
    wid = sid * _NC + cid
    pbase = wid * _PPW
    nbase = wid * _NPW

    cps = [
        pltpu.async_copy(pos_h.at[pl.ds(pbase, _PPW)], i_ph, semA),
        pltpu.async_copy(pos_t.at[pl.ds(pbase, _PPW)], i_pt, semA),
        pltpu.async_copy(pos_r.at[pl.ds(pbase, _PPW)], i_pr, semA),
        pltpu.async_copy(neg_h.at[pl.ds(nbase, _PPW)], i_nh0, semA),
        pltpu.async_copy(neg_h.at[pl.ds(nbase + _PPW, _PPW)], i_nh1, semA),
        pltpu.async_copy(neg_t.at[pl.ds(nbase, _PPW)], i_nt0, semA),
        pltpu.async_copy(neg_t.at[pl.ds(nbase + _PPW, _PPW)], i_nt1, semA),
        pltpu.async_copy(neg_r.at[pl.ds(nbase, _PPW)], i_nr0, semA),
        pltpu.async_copy(neg_r.at[pl.ds(nbase + _PPW, _PPW)], i_nr1, semA),
    ]
    for c in cps:
        c.wait()

    for i_ref, g_ref in ((i_ph, g_ph), (i_pt, g_pt), (i_pr, g_pr),
                         (i_nh0, g_nh0), (i_nh1, g_nh1), (i_nt0, g_nt0),
                         (i_nt1, g_nt1), (i_nr0, g_nr0), (i_nr1, g_nr1)):
        def shift_body(c, _, i_ref=i_ref, g_ref=g_ref):
            sl = pl.ds(c * _L, _L)
            g_ref[sl] = lax.shift_right_logical(i_ref[sl], 1)
            return 0
        lax.fori_loop(0, _PPW // _L, shift_body, 0)

    a_cps = [
        pltpu.async_copy(ent2.at[g_ph], bA0, semA),
        pltpu.async_copy(ent2.at[g_pt], bA1, semA),
        pltpu.async_copy(rel2.at[g_pr], bA2, semA),
    ]
    b_cps = [
        pltpu.async_copy(ent2.at[g_nh0], bB0, semB),
        pltpu.async_copy(ent2.at[g_nt0], bB1, semB),
        pltpu.async_copy(rel2.at[g_nr0], bB2, semB),
    ]
    for c in a_cps:
        c.wait()

    lane = lax.iota(jnp.int32, _L)
    zero = jnp.zeros((_L,), jnp.float32)

    accs_p = []
    for g in range(_GROUPS):
        rows = g * _L + lane
        par_h = (i_ph[pl.ds(g * _L, _L)] & 1) * EMBED_DIM
        par_t = (i_pt[pl.ds(g * _L, _L)] & 1) * EMBED_DIM
        par_r = (i_pr[pl.ds(g * _L, _L)] & 1) * EMBED_DIM

        def dbody(d, ap, rows=rows, par_h=par_h, par_t=par_t, par_r=par_r):
            h = plsc.load_gather(bA0, [rows, par_h + d])
            t = plsc.load_gather(bA1, [rows, par_t + d])
            r = plsc.load_gather(bA2, [rows, par_r + d])
            return ap + jnp.abs(h + r - t)

        accs_p.append(lax.fori_loop(0, EMBED_DIM, dbody, zero))

    a2_cps = [
        pltpu.async_copy(ent2.at[g_nh1], bA0, semA),
        pltpu.async_copy(ent2.at[g_nt1], bA1, semA),
        pltpu.async_copy(rel2.at[g_nr1], bA2, semA),
    ]
    for c in b_cps:
        c.wait()

    loss_acc = zero

    def neg_groups(g_lo, g_hi, ih, it, ir, b0, b1, b2, loss_acc):
        for g in range(g_lo, g_hi):
            j0 = 2 * ((g - g_lo) * _L + lane)
            j1 = j0 + 1
            ph0 = (plsc.load_gather(ih, [j0]) & 1) * EMBED_DIM
            pt0 = (plsc.load_gather(it, [j0]) & 1) * EMBED_DIM
            pr0 = (plsc.load_gather(ir, [j0]) & 1) * EMBED_DIM
            ph1 = (plsc.load_gather(ih, [j1]) & 1) * EMBED_DIM
            pt1 = (plsc.load_gather(it, [j1]) & 1) * EMBED_DIM
            pr1 = (plsc.load_gather(ir, [j1]) & 1) * EMBED_DIM

            def dbody(d, carry, j0=j0, j1=j1, ph0=ph0, pt0=pt0, pr0=pr0,
                      ph1=ph1, pt1=pt1, pr1=pr1, b0=b0, b1=b1, b2=b2):
                a0, a1 = carry
                h0 = plsc.load_gather(b0, [j0, ph0 + d])
                t0 = plsc.load_gather(b1, [j0, pt0 + d])
                r0 = plsc.load_gather(b2, [j0, pr0 + d])
                a0 = a0 + jnp.abs(h0 + r0 - t0)
                h1 = plsc.load_gather(b0, [j1, ph1 + d])
                t1 = plsc.load_gather(b1, [j1, pt1 + d])
                r1 = plsc.load_gather(b2, [j1, pr1 + d])
                a1 = a1 + jnp.abs(h1 + r1 - t1)
                return a0, a1

            an0, an1 = lax.fori_loop(0, EMBED_DIM, dbody, (zero, zero))
            n_sc = (an0 + an1) * 0.5
            loss_acc = loss_acc + jnp.maximum(accs_p[g] - n_sc + MARGIN, 0.0)
        return loss_acc

    loss_acc = neg_groups(0, _GROUPS // 2,
                          i_nh0, i_nt0, i_nr0, bB0, bB1, bB2, loss_acc)
    for c in a2_cps:
        c.wait()
    loss_acc = neg_groups(_GROUPS // 2, _GROUPS,
                          i_nh1, i_nt1, i_nr1, bA0, bA1, bA2, loss_acc)

    stage[pl.ds(0, _L)] = loss_acc
    pltpu.sync_copy(stage.at[pl.ds(0, _L)], shared.at[sid, pl.ds(0, _L)])
    plsc.subcore_barrier()

    @pl.when(sid == 0)
    def _():
        pltpu.sync_copy(shared, tmp)
        acc = jnp.zeros((_L,), jnp.float32)
        for w in range(_NS):
            acc = acc + tmp[w, pl.ds(0, _L)]
        total = jnp.sum(acc)
        stage[pl.ds(0, _L)] = jnp.full((_L,), total, jnp.float32)
        pltpu.sync_copy(stage.at[pl.ds(0, _L)], out.at[cid, pl.ds(0, _L)])


_FULLB = NUM_ENTITY // 128
_PARTIAL_COLS = NUM_ENTITY - _FULLB * 128
_MAIN_ITERS = (_FULLB // _NW) // 2
_TAIL_W = _FULLB - _MAIN_ITERS * 2 * _NW


def _transpose_body(entT, tail2, t2, ib0, ib1, ob0, ob1,
                    rsm0, rsm1, wsm0, wsm1):
    cid = lax.axis_index("c")
    sid = lax.axis_index("s")
    wid = sid * _NC + cid
    lane = lax.iota(jnp.int32, _L)

    def rd(buf, i, sm):
        c0 = pl.multiple_of((i * _NW + wid) * 128, 128)
        return pltpu.async_copy(entT.at[:, pl.ds(c0, 128)],
                                buf.at[:, pl.ds(0, 128)], sm)

    def wait_rd(buf, sm):
        pltpu.make_async_copy(entT.at[:, pl.ds(0, 128)],
                              buf.at[:, pl.ds(0, 128)], sm).wait()

    def transpose_block(src, ob):
        @plsc.parallel_loop(0, EMBED_DIM, unroll=4)
        def _(p):
            col0 = jnp.full((_L,), 2 * p, jnp.int32)
            col1 = col0 + 1
            for c in range(EMBED_DIM // _L):
                rows = c * _L + lane
                ob[p, pl.ds(c * _L, _L)] = plsc.load_gather(src,
                                                            [rows, col0])
                ob[p, pl.ds(EMBED_DIM + c * _L, _L)] = plsc.load_gather(
                    src, [rows, col1])

    def wr(ob, i, sm):
        p0 = pl.multiple_of((i * _NW + wid) * 64, 64)
        pltpu.async_copy(ob, t2.at[pl.ds(p0, 64)], sm)

    def wait_wr(ob, sm):
        pltpu.make_async_copy(ob, t2.at[pl.ds(0, 64)], sm).wait()

    rd(ib0, 0, rsm0)

    def pair(j, _):
        rd(ib1, 2 * j + 1, rsm1)
        wait_rd(ib0, rsm0)

        @pl.when(j > 0)
        def _():
            wait_wr(ob0, wsm0)

        transpose_block(ib0, ob0)
        wr(ob0, 2 * j, wsm0)

        @pl.when(j < _MAIN_ITERS - 1)
        def _():
            rd(ib0, 2 * j + 2, rsm0)

        wait_rd(ib1, rsm1)

        @pl.when(j > 0)
        def _():
            wait_wr(ob1, wsm1)

        transpose_block(ib1, ob1)
        wr(ob1, 2 * j + 1, wsm1)
        return 0

    lax.fori_loop(0, _MAIN_ITERS, pair, 0)
    wait_wr(ob0, wsm0)
    wait_wr(ob1, wsm1)

    @pl.when(wid < _TAIL_W)
    def _():
        bid = _MAIN_ITERS * 2 * _NW + wid
        c0 = pl.multiple_of(bid * 128, 128)
        pltpu.sync_copy(entT.at[:, pl.ds(c0, 128)], ib0.at[:, pl.ds(0, 128)])
        transpose_block(ib0, ob0)
        p0 = pl.multiple_of(bid * 64, 64)
        pltpu.sync_copy(ob0, t2.at[pl.ds(p0, 64)])

    @pl.when(wid == _TAIL_W)
    def _():
        nr = _PARTIAL_COLS // 2
        pltpu.sync_copy(tail2, ib0.at[pl.ds(0, nr), pl.ds(0, 128)])
        pltpu.sync_copy(ib0.at[pl.ds(0, nr), pl.ds(0, 128)],
                        t2.at[pl.ds(_FULLB * 64, nr)])


@jax.jit
def _transe_loss(pos_h, pos_r, pos_t, neg_h, neg_r, neg_t, entT, tail2, rel2):
    mesh = plsc.VectorSubcoreMesh(core_axis_name="c", subcore_axis_name="s")
    ent2 = pl.kernel(
        _transpose_body,
        out_type=jax.ShapeDtypeStruct((NUM_ENTITY // 2, PACK), jnp.float32),
        mesh=mesh,
        compiler_params=pltpu.CompilerParams(needs_layout_passes=False),
        scratch_types=[
            pltpu.VMEM((EMBED_DIM, 129), jnp.float32),
            pltpu.VMEM((EMBED_DIM, 129), jnp.float32),
            pltpu.VMEM((EMBED_DIM, PACK), jnp.float32),
            pltpu.VMEM((EMBED_DIM, PACK), jnp.float32),
            pltpu.SemaphoreType.DMA,
            pltpu.SemaphoreType.DMA,
            pltpu.SemaphoreType.DMA,
            pltpu.SemaphoreType.DMA,
        ],
    )(entT, tail2)
    idx_t = pltpu.VMEM((_PPW,), jnp.int32)
    buf_t = pltpu.VMEM((_PPW, PACK), jnp.float32)
    out = pl.kernel(
        _tec_body,
        out_type=jax.ShapeDtypeStruct((8, 128), jnp.float32),
        mesh=mesh,
        compiler_params=pltpu.CompilerParams(needs_layout_passes=False),
        scratch_types=[
            idx_t, idx_t, idx_t, idx_t, idx_t, idx_t, idx_t, idx_t, idx_t,
            idx_t, idx_t, idx_t, idx_t, idx_t, idx_t, idx_t, idx_t, idx_t,
            buf_t, buf_t, buf_t, buf_t, buf_t, buf_t,
            pltpu.VMEM_SHARED((_NS, 128), jnp.float32),
            pltpu.VMEM((_NS, 128), jnp.float32),
            pltpu.VMEM((128,), jnp.float32),
            pltpu.SemaphoreType.DMA,
            pltpu.SemaphoreType.DMA,
        ],
    )(pos_h, pos_r, pos_t, neg_h, neg_r, neg_t, ent2, rel2)
    return out[0, 0] + out[1, 0]


def kernel(pos_h, pos_r, pos_t, neg_h, neg_r, neg_t, entity_emb, relation_emb):
    rel2 = relation_emb.reshape(NUM_RELATION // 2, PACK)
    tail2 = entity_emb[_FULLB * 128:, :].reshape(_PARTIAL_COLS // 2, PACK)
    return _transe_loss(
        pos_h.astype(jnp.int32), pos_r.astype(jnp.int32),
        pos_t.astype(jnp.int32), neg_h.astype(jnp.int32),
        neg_r.astype(jnp.int32), neg_t.astype(jnp.int32),
        entity_emb.T, tail2, rel2)


# speedup vs baseline: 1.5182x; 1.3832x over previous
import jax
import jax.numpy as jnp
from jax import lax
from jax.experimental import pallas as pl
from jax.experimental.pallas import tpu as pltpu
from jax.experimental.pallas import tpu_sc as plsc

NUM_ENTITY = 1000000
NUM_RELATION = 1000
EMBED_DIM = 64
MARGIN = 1.0
NEG_SAMPLES = 2
BATCH = 4096

_INFO = plsc.get_sparse_core_info()
_NC = _INFO.num_cores
_NS = _INFO.num_subcores
_L = _INFO.num_lanes
_NW = _NC * _NS
_PPW = BATCH // _NW
_NPW = _PPW * NEG_SAMPLES


def _tec_body(pos_h, pos_r, pos_t, neg_h, neg_r, neg_t, ent, rel, out,
              i_ph, i_pt, i_pr, i_nh0, i_nh1, i_nt0, i_nt1, i_nr0, i_nr1,
              r_ph, r_pt, r_pr, r_nh0, r_nh1, r_nt0, r_nt1, r_nr0, r_nr1,
              shared, tmp, stage, sem):
    cid = lax.axis_index("c")
    sid = lax.axis_index("s")
    wid = sid * _NC + cid
    pbase = wid * _PPW
    nbase = wid * _NPW

    cps = [
        pltpu.async_copy(pos_h.at[pl.ds(pbase, _PPW)], i_ph, sem),
        pltpu.async_copy(pos_t.at[pl.ds(pbase, _PPW)], i_pt, sem),
        pltpu.async_copy(pos_r.at[pl.ds(pbase, _PPW)], i_pr, sem),
        pltpu.async_copy(neg_h.at[pl.ds(nbase, _PPW)], i_nh0, sem),
        pltpu.async_copy(neg_h.at[pl.ds(nbase + _PPW, _PPW)], i_nh1, sem),
        pltpu.async_copy(neg_t.at[pl.ds(nbase, _PPW)], i_nt0, sem),
        pltpu.async_copy(neg_t.at[pl.ds(nbase + _PPW, _PPW)], i_nt1, sem),
        pltpu.async_copy(neg_r.at[pl.ds(nbase, _PPW)], i_nr0, sem),
        pltpu.async_copy(neg_r.at[pl.ds(nbase + _PPW, _PPW)], i_nr1, sem),
    ]
    for c in cps:
        c.wait()

    cps = [
        pltpu.async_copy(ent.at[i_ph], r_ph, sem),
        pltpu.async_copy(ent.at[i_pt], r_pt, sem),
        pltpu.async_copy(rel.at[i_pr], r_pr, sem),
        pltpu.async_copy(ent.at[i_nh0], r_nh0, sem),
        pltpu.async_copy(ent.at[i_nh1], r_nh1, sem),
        pltpu.async_copy(ent.at[i_nt0], r_nt0, sem),
        pltpu.async_copy(ent.at[i_nt1], r_nt1, sem),
        pltpu.async_copy(rel.at[i_nr0], r_nr0, sem),
        pltpu.async_copy(rel.at[i_nr1], r_nr1, sem),
    ]
    for c in cps:
        c.wait()

    lane = lax.iota(jnp.int32, _L)
    zero = jnp.zeros((_L,), jnp.float32)

    def make_body(nh, nt, nr, off):
        def body(i, loss):
            j = 2 * (i - off)
            v = zero
            for c in range(EMBED_DIM // _L):
                sl = pl.ds(c * _L, _L)
                v = v + jnp.abs(r_ph[i, sl] + r_pr[i, sl] - r_pt[i, sl])
                v = v - 0.5 * jnp.abs(nh[j, sl] + nr[j, sl] - nt[j, sl])
                v = v - 0.5 * jnp.abs(nh[j + 1, sl] + nr[j + 1, sl]
                                      - nt[j + 1, sl])
            s = jnp.sum(v)
            return loss + jnp.maximum(s + MARGIN, 0.0)
        return body

    half = _PPW // 2
    loss = lax.fori_loop(0, half, make_body(r_nh0, r_nt0, r_nr0, 0), 0.0)
    loss = lax.fori_loop(half, _PPW,
                         make_body(r_nh1, r_nt1, r_nr1, half), loss)
    loss_acc = jnp.where(lane == 0, loss, 0.0)

    stage[...] = loss_acc
    pltpu.sync_copy(stage, shared.at[sid])
    plsc.subcore_barrier()

    @pl.when(sid == 0)
    def _():
        pltpu.sync_copy(shared, tmp)
        acc = jnp.zeros((_L,), jnp.float32)
        for w in range(_NS):
            acc = acc + tmp[w]
        total = jnp.sum(acc)
        stage[...] = jnp.full((_L,), total, jnp.float32)
        pltpu.sync_copy(stage, out.at[cid])


@jax.jit
def _transe_loss(pos_h, pos_r, pos_t, neg_h, neg_r, neg_t, ent, rel):
    mesh = plsc.VectorSubcoreMesh(core_axis_name="c", subcore_axis_name="s")
    idx_t = pltpu.VMEM((_PPW,), jnp.int32)
    row_t = pltpu.VMEM((_PPW, EMBED_DIM), jnp.float32)
    out = pl.kernel(
        _tec_body,
        out_type=jax.ShapeDtypeStruct((_NC, _L), jnp.float32),
        mesh=mesh,
        compiler_params=pltpu.CompilerParams(
            needs_layout_passes=False, use_tc_tiling_on_sc=False),
        scratch_types=[
            idx_t, idx_t, idx_t, idx_t, idx_t, idx_t, idx_t, idx_t, idx_t,
            row_t, row_t, row_t, row_t, row_t, row_t, row_t, row_t, row_t,
            pltpu.VMEM_SHARED((_NS, _L), jnp.float32),
            pltpu.VMEM((_NS, _L), jnp.float32),
            pltpu.VMEM((_L,), jnp.float32),
            pltpu.SemaphoreType.DMA,
        ],
    )(pos_h, pos_r, pos_t, neg_h, neg_r, neg_t, ent, rel)
    return out[0, 0] + out[1, 0]


def kernel(pos_h, pos_r, pos_t, neg_h, neg_r, neg_t, entity_emb, relation_emb):
    return _transe_loss(
        pos_h.astype(jnp.int32), pos_r.astype(jnp.int32),
        pos_t.astype(jnp.int32), neg_h.astype(jnp.int32),
        neg_r.astype(jnp.int32), neg_t.astype(jnp.int32),
        entity_emb, relation_emb)
